# baseline (speedup 1.0000x reference)
import functools

import jax
import jax.numpy as jnp
from jax import lax
from jax.experimental import pallas as pl
from jax.experimental.pallas import tpu as pltpu
from jax.experimental.pallas import tpu_sc as plsc

N = 10000
E = 320000
R = 8
DH = 256
HC = 128
NC = 2
NS = 16
NW = NC * NS
TPW = 5120
CNT_PAD = NS * TPW
YROWS = NC * R * N

CH = 2000
WIN = 80
NPAD = 10240


def _mesh():
    return plsc.VectorSubcoreMesh(
        core_axis_name="c", subcore_axis_name="s",
        num_cores=NC, num_subcores=NS)


def _prep0_body(src_ref, dst_ref, typ_ref, gidx_ref, sidx_ref):
    gidx_ref[...] = typ_ref[...] * N + src_ref[...]
    sidx_ref[...] = dst_ref[...] * R + typ_ref[...]


def _prep0(src, dst, typ):
    shp = (E // 128, 128)
    gidx, sidx = pl.pallas_call(
        _prep0_body,
        out_shape=(jax.ShapeDtypeStruct(shp, jnp.int32),
                   jax.ShapeDtypeStruct(shp, jnp.int32)),
    )(src.reshape(shp), dst.reshape(shp), typ.reshape(shp))
    return gidx.reshape(-1), sidx.reshape(-1)


def _sc_count(sidx):
    EPW = E // NW

    @functools.partial(
        pl.kernel, mesh=_mesh(),
        out_type=jax.ShapeDtypeStruct((NC, CNT_PAD), jnp.float32),
        scratch_types=[
            pltpu.VMEM_SHARED((CNT_PAD,), jnp.float32),
            pltpu.VMEM((TPW,), jnp.float32),
            pltpu.VMEM((CH,), jnp.int32),
            pltpu.VMEM((WIN,), jnp.int32),
            pltpu.VMEM((WIN,), jnp.float32),
        ])
    def body(sidx_hbm, out_hbm, cnt_sh, zbuf, sidx_big, sidx_w, ones_v):
        c = lax.axis_index("c")
        s = lax.axis_index("s")
        wid = s * NC + c

        @pl.loop(0, TPW // 16)
        def _(i):
            zbuf[pl.ds(i * 16, 16)] = jnp.zeros((16,), jnp.float32)

        @pl.loop(0, WIN // 16)
        def _(i):
            ones_v[pl.ds(i * 16, 16)] = jnp.ones((16,), jnp.float32)

        pltpu.sync_copy(zbuf, cnt_sh.at[pl.ds(s * TPW, TPW)])
        plsc.subcore_barrier()

        @pl.loop(0, EPW // CH)
        def _(o):
            base = wid * EPW + o * CH
            pltpu.sync_copy(sidx_hbm.at[pl.ds(base, CH)], sidx_big)

            @pl.loop(0, CH // WIN)
            def _(i):
                @pl.loop(0, WIN // 16)
                def _(q):
                    sidx_w[pl.ds(q * 16, 16)] = (
                        sidx_big[pl.ds(i * WIN + q * 16, 16)])
                pltpu.sync_copy(ones_v, cnt_sh.at[sidx_w], add=True)

        plsc.subcore_barrier()
        pltpu.sync_copy(cnt_sh.at[pl.ds(s * TPW, TPW)],
                        out_hbm.at[c, pl.ds(s * TPW, TPW)])

    return body(sidx)


def _prep1_body(cnt_ref, inv_ref):
    tot = cnt_ref[0] + cnt_ref[1]
    inv_ref[...] = 1.0 / jnp.maximum(tot, 1.0)


def _prep1(cntp):
    out = pl.pallas_call(
        _prep1_body,
        out_shape=jax.ShapeDtypeStruct((CNT_PAD // 128, 128), jnp.float32),
    )(cntp.reshape(NC, CNT_PAD // 128, 128))
    return out.reshape(-1)


def _sc_w(inv, sidx):
    EPW = E // NW

    @functools.partial(
        pl.kernel, mesh=_mesh(),
        out_type=jax.ShapeDtypeStruct((E,), jnp.float32),
        scratch_types=[
            pltpu.VMEM((CH,), jnp.int32),
            pltpu.VMEM((CH,), jnp.float32),
            pltpu.VMEM((WIN,), jnp.int32),
            pltpu.VMEM((WIN,), jnp.int32),
            pltpu.VMEM((WIN,), jnp.float32),
            pltpu.VMEM((WIN,), jnp.float32),
            pltpu.SemaphoreType.DMA,
            pltpu.SemaphoreType.DMA,
        ])
    def body(inv_hbm, sidx_hbm, w_hbm, sidx_big, w_big,
             sidx_wa, sidx_wb, w_wa, w_wb, sem_a, sem_b):
        c = lax.axis_index("c")
        s = lax.axis_index("s")
        wid = s * NC + c
        NWC = CH // WIN

        def start(i, sidx_w, w_win, sem):
            @pl.loop(0, WIN // 16)
            def _(q):
                sidx_w[pl.ds(q * 16, 16)] = (
                    sidx_big[pl.ds(i * WIN + q * 16, 16)])
            pltpu.async_copy(inv_hbm.at[sidx_w], w_win, sem)

        def finish(i, sidx_w, w_win, sem):
            pltpu.make_async_copy(inv_hbm.at[sidx_w], w_win, sem).wait()

            @pl.loop(0, WIN // 16)
            def _(q):
                w_big[pl.ds(i * WIN + q * 16, 16)] = (
                    w_win[pl.ds(q * 16, 16)])

        @pl.loop(0, EPW // CH)
        def _(o):
            base = wid * EPW + o * CH
            pltpu.sync_copy(sidx_hbm.at[pl.ds(base, CH)], sidx_big)

            start(0, sidx_wa, w_wa, sem_a)

            @pl.loop(0, (NWC - 1) // 2)
            def _(k):
                i0 = 2 * k
                start(i0 + 1, sidx_wb, w_wb, sem_b)
                finish(i0, sidx_wa, w_wa, sem_a)
                start(i0 + 2, sidx_wa, w_wa, sem_a)
                finish(i0 + 1, sidx_wb, w_wb, sem_b)

            finish(NWC - 1, sidx_wa, w_wa, sem_a)
            pltpu.sync_copy(w_big, w_hbm.at[pl.ds(base, CH)])

    return body(inv, sidx)


def _y_transform(h, W):
    d_in = h.shape[1]
    BN = 1000
    NB = N // BN

    def body(h_ref, w_ref, y_ref):
        y_ref[...] = jnp.dot(h_ref[...], w_ref[0],
                             preferred_element_type=jnp.float32)

    return pl.pallas_call(
        body,
        grid=(NB, NC, R),
        in_specs=[
            pl.BlockSpec((BN, d_in), lambda nb, c, r: (nb, 0)),
            pl.BlockSpec((1, d_in, HC), lambda nb, c, r: (r, 0, c)),
        ],
        out_specs=pl.BlockSpec((BN, HC),
                               lambda nb, c, r: (c * R * NB + r * NB + nb, 0)),
        out_shape=jax.ShapeDtypeStruct((YROWS, HC), jnp.float32),
    )(h, W)


def _sc_agg(y, gidx, dst, w):
    EPT = E // NS
    RPT = NPAD // NS
    ZR = 128

    @functools.partial(
        pl.kernel, mesh=_mesh(),
        out_type=jax.ShapeDtypeStruct((NC, NPAD, HC), jnp.float32),
        scratch_types=[
            pltpu.VMEM_SHARED((NPAD, HC), jnp.float32),
            pltpu.VMEM((WIN, HC), jnp.float32),
            pltpu.VMEM((WIN, HC), jnp.float32),
            pltpu.VMEM((CH,), jnp.int32),
            pltpu.VMEM((CH,), jnp.int32),
            pltpu.VMEM((CH,), jnp.float32),
            pltpu.VMEM((WIN,), jnp.int32),
            pltpu.VMEM((WIN,), jnp.int32),
            pltpu.VMEM((WIN,), jnp.int32),
            pltpu.VMEM((WIN,), jnp.int32),
            pltpu.VMEM((ZR, HC), jnp.float32),
            pltpu.SemaphoreType.DMA,
            pltpu.SemaphoreType.DMA,
            pltpu.SemaphoreType.DMA,
            pltpu.SemaphoreType.DMA,
        ])
    def body(y_hbm, gidx_hbm, dst_hbm, w_hbm, msg_hbm,
             acc, rows_a, rows_b, gidx_big, dst_big, w_big,
             gidx_a, gidx_b, dst_a, dst_b, zbuf, sem_a, sem_b,
             ssem_a, ssem_b):
        c = lax.axis_index("c")
        s = lax.axis_index("s")

        @pl.loop(0, ZR)
        def _(i):
            for q in range(HC // 16):
                zbuf[i, pl.ds(q * 16, 16)] = jnp.zeros((16,), jnp.float32)

        @pl.loop(0, RPT // ZR)
        def _(k):
            pltpu.sync_copy(zbuf, acc.at[pl.ds(s * RPT + k * ZR, ZR)])

        plsc.subcore_barrier()
        off = c * (R * N)
        NWC = CH // WIN

        def start(i, gidx_w, dst_w, rows, sem):
            @pl.loop(0, WIN // 16)
            def _(q):
                gidx_w[pl.ds(q * 16, 16)] = (
                    gidx_big[pl.ds(i * WIN + q * 16, 16)] + off)
                dst_w[pl.ds(q * 16, 16)] = (
                    dst_big[pl.ds(i * WIN + q * 16, 16)])
            pltpu.async_copy(y_hbm.at[gidx_w], rows, sem)

        def scale_scatter(i, gidx_w, dst_w, rows, sem, ssem):
            pltpu.make_async_copy(y_hbm.at[gidx_w], rows, sem).wait()

            @pl.loop(0, WIN // 16)
            def _(jj):
                wv = w_big[pl.ds(i * WIN + jj * 16, 16)]
                for l in range(16):
                    wj = wv[l]
                    for q in range(HC // 16):
                        rows[jj * 16 + l, pl.ds(q * 16, 16)] = (
                            rows[jj * 16 + l, pl.ds(q * 16, 16)] * wj)

            pltpu.async_copy(rows, acc.at[dst_w], ssem, add=True)

        def wait_scatter(dst_w, rows, ssem):
            pltpu.make_async_copy(rows, acc.at[dst_w], ssem).wait()

        @pl.loop(0, EPT // CH)
        def _(o):
            base = s * EPT + o * CH
            pltpu.sync_copy(gidx_hbm.at[pl.ds(base, CH)], gidx_big)
            pltpu.sync_copy(dst_hbm.at[pl.ds(base, CH)], dst_big)
            pltpu.sync_copy(w_hbm.at[pl.ds(base, CH)], w_big)

            start(0, gidx_a, dst_a, rows_a, sem_a)
            start(1, gidx_b, dst_b, rows_b, sem_b)

            @pl.loop(0, (NWC - 1) // 2)
            def _(k):
                i0 = 2 * k
                scale_scatter(i0, gidx_a, dst_a, rows_a, sem_a, ssem_a)
                scale_scatter(i0 + 1, gidx_b, dst_b, rows_b, sem_b, ssem_b)
                wait_scatter(dst_a, rows_a, ssem_a)
                start(i0 + 2, gidx_a, dst_a, rows_a, sem_a)
                wait_scatter(dst_b, rows_b, ssem_b)

                @pl.when(i0 + 3 < NWC)
                def _():
                    start(i0 + 3, gidx_b, dst_b, rows_b, sem_b)

            scale_scatter(NWC - 1, gidx_a, dst_a, rows_a, sem_a, ssem_a)
            wait_scatter(dst_a, rows_a, ssem_a)

        plsc.subcore_barrier()

        @pl.loop(0, RPT // ZR)
        def _(k):
            r0 = s * RPT + k * ZR
            pltpu.sync_copy(acc.at[pl.ds(r0, ZR)],
                            msg_hbm.at[c, pl.ds(r0, ZR)])

    return body(y, gidx, dst, w)


def _tc_out(h, root, b, msg):
    d_in = h.shape[1]
    BN = 1000
    NB = N // BN

    def body(h_ref, root_ref, b_ref, msg_ref, z_ref):
        t = jnp.dot(h_ref[...], root_ref[...],
                    preferred_element_type=jnp.float32)
        z_ref[...] = jnp.maximum(t + b_ref[0] + msg_ref[0], 0.0)

    return pl.pallas_call(
        body,
        grid=(NB, NC),
        in_specs=[
            pl.BlockSpec((BN, d_in), lambda nb, c: (nb, 0)),
            pl.BlockSpec((d_in, HC), lambda nb, c: (0, c)),
            pl.BlockSpec((1, 1, HC), lambda nb, c: (c, 0, 0)),
            pl.BlockSpec((1, BN, HC), lambda nb, c: (c, nb, 0)),
        ],
        out_specs=pl.BlockSpec((BN, HC), lambda nb, c: (nb, c)),
        out_shape=jax.ShapeDtypeStruct((N, DH), jnp.float32),
    )(h, root, b.reshape(NC, 1, HC), msg)


def kernel(x, edge_index, edge_type, W1, root1, b1, W2, root2, b2):
    src = edge_index[0].astype(jnp.int32)
    dst = edge_index[1].astype(jnp.int32)
    typ = edge_type.astype(jnp.int32)

    gidx, sidx = _prep0(src, dst, typ)
    cntp = _sc_count(sidx)
    inv = _prep1(cntp)
    w = _sc_w(inv, sidx)

    y1 = _y_transform(x, W1)
    msg1 = _sc_agg(y1, gidx, dst, w)
    z = _tc_out(x, root1, b1, msg1)

    y2 = _y_transform(z, W2)
    msg2 = _sc_agg(y2, gidx, dst, w)
    out = _tc_out(z, root2, b2, msg2)
    return out


# speedup vs baseline: 1.0640x; 1.0640x over previous
import functools

import jax
import jax.numpy as jnp
from jax import lax
from jax.experimental import pallas as pl
from jax.experimental.pallas import tpu as pltpu
from jax.experimental.pallas import tpu_sc as plsc

N = 10000
E = 320000
R = 8
DH = 256
HC = 128
NC = 2
NS = 16
NW = NC * NS
TPW = 5120
CNT_PAD = NS * TPW
YROWS = NC * R * N

CH = 2000
WIN = 80
NPAD = 10240


def _mesh():
    return plsc.VectorSubcoreMesh(
        core_axis_name="c", subcore_axis_name="s",
        num_cores=NC, num_subcores=NS)


def _prep0_body(src_ref, dst_ref, typ_ref, gidx_ref, sidx_ref):
    gidx_ref[...] = typ_ref[...] * N + src_ref[...]
    sidx_ref[...] = dst_ref[...] * R + typ_ref[...]


def _prep0(src, dst, typ):
    shp = (E // 128, 128)
    gidx, sidx = pl.pallas_call(
        _prep0_body,
        out_shape=(jax.ShapeDtypeStruct(shp, jnp.int32),
                   jax.ShapeDtypeStruct(shp, jnp.int32)),
    )(src.reshape(shp), dst.reshape(shp), typ.reshape(shp))
    return gidx.reshape(-1), sidx.reshape(-1)


def _sc_count(sidx):
    EPW = E // NW

    @functools.partial(
        pl.kernel, mesh=_mesh(),
        out_type=jax.ShapeDtypeStruct((NC, CNT_PAD), jnp.float32),
        scratch_types=[
            pltpu.VMEM_SHARED((CNT_PAD,), jnp.float32),
            pltpu.VMEM((TPW,), jnp.float32),
            pltpu.VMEM((CH,), jnp.int32),
            pltpu.VMEM((WIN,), jnp.int32),
            pltpu.VMEM((WIN,), jnp.float32),
        ])
    def body(sidx_hbm, out_hbm, cnt_sh, zbuf, sidx_big, sidx_w, ones_v):
        c = lax.axis_index("c")
        s = lax.axis_index("s")
        wid = s * NC + c

        @pl.loop(0, TPW // 16)
        def _(i):
            zbuf[pl.ds(i * 16, 16)] = jnp.zeros((16,), jnp.float32)

        @pl.loop(0, WIN // 16)
        def _(i):
            ones_v[pl.ds(i * 16, 16)] = jnp.ones((16,), jnp.float32)

        pltpu.sync_copy(zbuf, cnt_sh.at[pl.ds(s * TPW, TPW)])
        plsc.subcore_barrier()

        @pl.loop(0, EPW // CH)
        def _(o):
            base = wid * EPW + o * CH
            pltpu.sync_copy(sidx_hbm.at[pl.ds(base, CH)], sidx_big)

            @pl.loop(0, CH // WIN)
            def _(i):
                @pl.loop(0, WIN // 16)
                def _(q):
                    sidx_w[pl.ds(q * 16, 16)] = (
                        sidx_big[pl.ds(i * WIN + q * 16, 16)])
                pltpu.sync_copy(ones_v, cnt_sh.at[sidx_w], add=True)

        plsc.subcore_barrier()
        pltpu.sync_copy(cnt_sh.at[pl.ds(s * TPW, TPW)],
                        out_hbm.at[c, pl.ds(s * TPW, TPW)])

    return body(sidx)


def _prep1_body(cnt_ref, inv_ref):
    tot = cnt_ref[0] + cnt_ref[1]
    inv_ref[...] = 1.0 / jnp.maximum(tot, 1.0)


def _prep1(cntp):
    out = pl.pallas_call(
        _prep1_body,
        out_shape=jax.ShapeDtypeStruct((CNT_PAD // 128, 128), jnp.float32),
    )(cntp.reshape(NC, CNT_PAD // 128, 128))
    return out.reshape(-1)


def _sc_w(inv, sidx):
    EPW = E // NW

    @functools.partial(
        pl.kernel, mesh=_mesh(),
        out_type=jax.ShapeDtypeStruct((E,), jnp.float32),
        scratch_types=[
            pltpu.VMEM((CH,), jnp.int32),
            pltpu.VMEM((CH,), jnp.float32),
            pltpu.VMEM((WIN,), jnp.int32),
            pltpu.VMEM((WIN,), jnp.int32),
            pltpu.VMEM((WIN,), jnp.float32),
            pltpu.VMEM((WIN,), jnp.float32),
            pltpu.SemaphoreType.DMA,
            pltpu.SemaphoreType.DMA,
        ])
    def body(inv_hbm, sidx_hbm, w_hbm, sidx_big, w_big,
             sidx_wa, sidx_wb, w_wa, w_wb, sem_a, sem_b):
        c = lax.axis_index("c")
        s = lax.axis_index("s")
        wid = s * NC + c
        NWC = CH // WIN

        def start(i, sidx_w, w_win, sem):
            @pl.loop(0, WIN // 16)
            def _(q):
                sidx_w[pl.ds(q * 16, 16)] = (
                    sidx_big[pl.ds(i * WIN + q * 16, 16)])
            pltpu.async_copy(inv_hbm.at[sidx_w], w_win, sem)

        def finish(i, sidx_w, w_win, sem):
            pltpu.make_async_copy(inv_hbm.at[sidx_w], w_win, sem).wait()

            @pl.loop(0, WIN // 16)
            def _(q):
                w_big[pl.ds(i * WIN + q * 16, 16)] = (
                    w_win[pl.ds(q * 16, 16)])

        @pl.loop(0, EPW // CH)
        def _(o):
            base = wid * EPW + o * CH
            pltpu.sync_copy(sidx_hbm.at[pl.ds(base, CH)], sidx_big)

            start(0, sidx_wa, w_wa, sem_a)

            @pl.loop(0, (NWC - 1) // 2)
            def _(k):
                i0 = 2 * k
                start(i0 + 1, sidx_wb, w_wb, sem_b)
                finish(i0, sidx_wa, w_wa, sem_a)
                start(i0 + 2, sidx_wa, w_wa, sem_a)
                finish(i0 + 1, sidx_wb, w_wb, sem_b)

            finish(NWC - 1, sidx_wa, w_wa, sem_a)
            pltpu.sync_copy(w_big, w_hbm.at[pl.ds(base, CH)])

    return body(inv, sidx)


def _y_transform(h, W):
    d_in = h.shape[1]
    BN = 1000
    NB = N // BN

    def body(h_ref, w_ref, y_ref):
        y_ref[...] = jnp.dot(h_ref[...], w_ref[0],
                             preferred_element_type=jnp.float32)

    return pl.pallas_call(
        body,
        grid=(NB, NC, R),
        in_specs=[
            pl.BlockSpec((BN, d_in), lambda nb, c, r: (nb, 0)),
            pl.BlockSpec((1, d_in, HC), lambda nb, c, r: (r, 0, c)),
        ],
        out_specs=pl.BlockSpec((BN, HC),
                               lambda nb, c, r: (c * R * NB + r * NB + nb, 0)),
        out_shape=jax.ShapeDtypeStruct((YROWS, HC), jnp.float32),
    )(h, W)


def _sc_agg(y, gidx, dst, w):
    EPT = E // NS
    RPT = NPAD // NS
    ZR = 128

    @functools.partial(
        pl.kernel, mesh=_mesh(),
        out_type=jax.ShapeDtypeStruct((NC, NPAD, HC), jnp.float32),
        scratch_types=[
            pltpu.VMEM_SHARED((NPAD, HC), jnp.float32),
            pltpu.VMEM((WIN, HC), jnp.float32),
            pltpu.VMEM((WIN, HC), jnp.float32),
            pltpu.VMEM((CH,), jnp.int32),
            pltpu.VMEM((CH,), jnp.int32),
            pltpu.VMEM((CH,), jnp.float32),
            pltpu.VMEM((WIN,), jnp.int32),
            pltpu.VMEM((WIN,), jnp.int32),
            pltpu.VMEM((WIN,), jnp.int32),
            pltpu.VMEM((WIN,), jnp.int32),
            pltpu.VMEM((ZR, HC), jnp.float32),
            pltpu.SemaphoreType.DMA,
            pltpu.SemaphoreType.DMA,
        ])
    def body(y_hbm, gidx_hbm, dst_hbm, w_hbm, msg_hbm,
             acc, rows_a, rows_b, gidx_big, dst_big, w_big,
             gidx_a, gidx_b, dst_a, dst_b, zbuf, sem_a, sem_b):
        c = lax.axis_index("c")
        s = lax.axis_index("s")

        @pl.loop(0, ZR)
        def _(i):
            for q in range(HC // 16):
                zbuf[i, pl.ds(q * 16, 16)] = jnp.zeros((16,), jnp.float32)

        @pl.loop(0, RPT // ZR)
        def _(k):
            pltpu.sync_copy(zbuf, acc.at[pl.ds(s * RPT + k * ZR, ZR)])

        plsc.subcore_barrier()
        off = c * (R * N)
        NWC = CH // WIN

        def start(i, gidx_w, dst_w, rows, sem):
            @pl.loop(0, WIN // 16)
            def _(q):
                gidx_w[pl.ds(q * 16, 16)] = (
                    gidx_big[pl.ds(i * WIN + q * 16, 16)] + off)
                dst_w[pl.ds(q * 16, 16)] = (
                    dst_big[pl.ds(i * WIN + q * 16, 16)])
            pltpu.async_copy(y_hbm.at[gidx_w], rows, sem)

        def finish(i, gidx_w, dst_w, rows, sem):
            pltpu.make_async_copy(y_hbm.at[gidx_w], rows, sem).wait()

            @pl.loop(0, WIN // 16)
            def _(jj):
                wv = w_big[pl.ds(i * WIN + jj * 16, 16)]
                for l in range(16):
                    wj = wv[l]
                    for q in range(HC // 16):
                        rows[jj * 16 + l, pl.ds(q * 16, 16)] = (
                            rows[jj * 16 + l, pl.ds(q * 16, 16)] * wj)

            pltpu.sync_copy(rows, acc.at[dst_w], add=True)

        @pl.loop(0, EPT // CH)
        def _(o):
            base = s * EPT + o * CH
            pltpu.sync_copy(gidx_hbm.at[pl.ds(base, CH)], gidx_big)
            pltpu.sync_copy(dst_hbm.at[pl.ds(base, CH)], dst_big)
            pltpu.sync_copy(w_hbm.at[pl.ds(base, CH)], w_big)

            start(0, gidx_a, dst_a, rows_a, sem_a)

            @pl.loop(0, (NWC - 1) // 2)
            def _(k):
                i0 = 2 * k
                start(i0 + 1, gidx_b, dst_b, rows_b, sem_b)
                finish(i0, gidx_a, dst_a, rows_a, sem_a)
                start(i0 + 2, gidx_a, dst_a, rows_a, sem_a)
                finish(i0 + 1, gidx_b, dst_b, rows_b, sem_b)

            finish(NWC - 1, gidx_a, dst_a, rows_a, sem_a)

        plsc.subcore_barrier()

        @pl.loop(0, RPT // ZR)
        def _(k):
            r0 = s * RPT + k * ZR
            pltpu.sync_copy(acc.at[pl.ds(r0, ZR)],
                            msg_hbm.at[c, pl.ds(r0, ZR)])

    return body(y, gidx, dst, w)


def _tc_out(h, root, b, msg):
    d_in = h.shape[1]
    BN = 1000
    NB = N // BN

    def body(h_ref, root_ref, b_ref, msg_ref, z_ref):
        t = jnp.dot(h_ref[...], root_ref[...],
                    preferred_element_type=jnp.float32)
        z_ref[...] = jnp.maximum(t + b_ref[0] + msg_ref[0], 0.0)

    return pl.pallas_call(
        body,
        grid=(NB, NC),
        in_specs=[
            pl.BlockSpec((BN, d_in), lambda nb, c: (nb, 0)),
            pl.BlockSpec((d_in, HC), lambda nb, c: (0, c)),
            pl.BlockSpec((1, 1, HC), lambda nb, c: (c, 0, 0)),
            pl.BlockSpec((1, BN, HC), lambda nb, c: (c, nb, 0)),
        ],
        out_specs=pl.BlockSpec((BN, HC), lambda nb, c: (nb, c)),
        out_shape=jax.ShapeDtypeStruct((N, DH), jnp.float32),
    )(h, root, b.reshape(NC, 1, HC), msg)


def kernel(x, edge_index, edge_type, W1, root1, b1, W2, root2, b2):
    src = edge_index[0].astype(jnp.int32)
    dst = edge_index[1].astype(jnp.int32)
    typ = edge_type.astype(jnp.int32)

    gidx, sidx = _prep0(src, dst, typ)
    cntp = _sc_count(sidx)
    inv = _prep1(cntp)
    w = _sc_w(inv, sidx)

    y1 = _y_transform(x, W1)
    msg1 = _sc_agg(y1, gidx, dst, w)
    z = _tc_out(x, root1, b1, msg1)

    y2 = _y_transform(z, W2)
    msg2 = _sc_agg(y2, gidx, dst, w)
    out = _tc_out(z, root2, b2, msg2)
    return out


# speedup vs baseline: 1.0761x; 1.0114x over previous
import functools

import jax
import jax.numpy as jnp
from jax import lax
from jax.experimental import pallas as pl
from jax.experimental.pallas import tpu as pltpu
from jax.experimental.pallas import tpu_sc as plsc

N = 10000
E = 320000
R = 8
DH = 256
HC = 128
NC = 2
NS = 16
NW = NC * NS
TPW = 5120
CNT_PAD = NS * TPW
YROWS = NC * R * N

CH = 2000
WIN = 80
NPAD = 10240
EPAD = 327680
CH2 = 2048
WIN2 = 128


def _mesh():
    return plsc.VectorSubcoreMesh(
        core_axis_name="c", subcore_axis_name="s",
        num_cores=NC, num_subcores=NS)


def _prep0_body(src_ref, dst_ref, typ_ref, gidx_ref, sidx_ref):
    gidx_ref[...] = typ_ref[...] * N + src_ref[...]
    sidx_ref[...] = dst_ref[...] * R + typ_ref[...]


def _prep0(src, dst, typ):
    shp = (E // 128, 128)
    gidx, sidx = pl.pallas_call(
        _prep0_body,
        out_shape=(jax.ShapeDtypeStruct(shp, jnp.int32),
                   jax.ShapeDtypeStruct(shp, jnp.int32)),
    )(src.reshape(shp), dst.reshape(shp), typ.reshape(shp))
    return gidx.reshape(-1), sidx.reshape(-1)


def _sc_count(sidx):
    EPW = E // NW

    @functools.partial(
        pl.kernel, mesh=_mesh(),
        out_type=jax.ShapeDtypeStruct((NC, CNT_PAD), jnp.float32),
        scratch_types=[
            pltpu.VMEM_SHARED((CNT_PAD,), jnp.float32),
            pltpu.VMEM((TPW,), jnp.float32),
            pltpu.VMEM((CH,), jnp.int32),
            pltpu.VMEM((WIN,), jnp.int32),
            pltpu.VMEM((WIN,), jnp.float32),
        ])
    def body(sidx_hbm, out_hbm, cnt_sh, zbuf, sidx_big, sidx_w, ones_v):
        c = lax.axis_index("c")
        s = lax.axis_index("s")
        wid = s * NC + c

        @pl.loop(0, TPW // 16)
        def _(i):
            zbuf[pl.ds(i * 16, 16)] = jnp.zeros((16,), jnp.float32)

        @pl.loop(0, WIN // 16)
        def _(i):
            ones_v[pl.ds(i * 16, 16)] = jnp.ones((16,), jnp.float32)

        pltpu.sync_copy(zbuf, cnt_sh.at[pl.ds(s * TPW, TPW)])
        plsc.subcore_barrier()

        @pl.loop(0, EPW // CH)
        def _(o):
            base = wid * EPW + o * CH
            pltpu.sync_copy(sidx_hbm.at[pl.ds(base, CH)], sidx_big)

            @pl.loop(0, CH // WIN)
            def _(i):
                @pl.loop(0, WIN // 16)
                def _(q):
                    sidx_w[pl.ds(q * 16, 16)] = (
                        sidx_big[pl.ds(i * WIN + q * 16, 16)])
                pltpu.sync_copy(ones_v, cnt_sh.at[sidx_w], add=True)

        plsc.subcore_barrier()
        pltpu.sync_copy(cnt_sh.at[pl.ds(s * TPW, TPW)],
                        out_hbm.at[c, pl.ds(s * TPW, TPW)])

    return body(sidx)


def _prep1_body(cnt_ref, inv_ref):
    tot = cnt_ref[0] + cnt_ref[1]
    inv_ref[...] = 1.0 / jnp.maximum(tot, 1.0)


def _prep1(cntp):
    out = pl.pallas_call(
        _prep1_body,
        out_shape=jax.ShapeDtypeStruct((CNT_PAD // 128, 128), jnp.float32),
    )(cntp.reshape(NC, CNT_PAD // 128, 128))
    return out.reshape(-1)


def _sc_w(inv, sidx):
    EPW = E // NW

    @functools.partial(
        pl.kernel, mesh=_mesh(),
        out_type=jax.ShapeDtypeStruct((E,), jnp.float32),
        scratch_types=[
            pltpu.VMEM((CH,), jnp.int32),
            pltpu.VMEM((CH,), jnp.float32),
            pltpu.VMEM((WIN,), jnp.int32),
            pltpu.VMEM((WIN,), jnp.int32),
            pltpu.VMEM((WIN,), jnp.float32),
            pltpu.VMEM((WIN,), jnp.float32),
            pltpu.SemaphoreType.DMA,
            pltpu.SemaphoreType.DMA,
        ])
    def body(inv_hbm, sidx_hbm, w_hbm, sidx_big, w_big,
             sidx_wa, sidx_wb, w_wa, w_wb, sem_a, sem_b):
        c = lax.axis_index("c")
        s = lax.axis_index("s")
        wid = s * NC + c
        NWC = CH // WIN

        def start(i, sidx_w, w_win, sem):
            @pl.loop(0, WIN // 16)
            def _(q):
                sidx_w[pl.ds(q * 16, 16)] = (
                    sidx_big[pl.ds(i * WIN + q * 16, 16)])
            pltpu.async_copy(inv_hbm.at[sidx_w], w_win, sem)

        def finish(i, sidx_w, w_win, sem):
            pltpu.make_async_copy(inv_hbm.at[sidx_w], w_win, sem).wait()

            @pl.loop(0, WIN // 16)
            def _(q):
                w_big[pl.ds(i * WIN + q * 16, 16)] = (
                    w_win[pl.ds(q * 16, 16)])

        @pl.loop(0, EPW // CH)
        def _(o):
            base = wid * EPW + o * CH
            pltpu.sync_copy(sidx_hbm.at[pl.ds(base, CH)], sidx_big)

            start(0, sidx_wa, w_wa, sem_a)

            @pl.loop(0, (NWC - 1) // 2)
            def _(k):
                i0 = 2 * k
                start(i0 + 1, sidx_wb, w_wb, sem_b)
                finish(i0, sidx_wa, w_wa, sem_a)
                start(i0 + 2, sidx_wa, w_wa, sem_a)
                finish(i0 + 1, sidx_wb, w_wb, sem_b)

            finish(NWC - 1, sidx_wa, w_wa, sem_a)
            pltpu.sync_copy(w_big, w_hbm.at[pl.ds(base, CH)])

    return body(inv, sidx)


def _y_transform(h, W):
    d_in = h.shape[1]
    BN = 1000
    NB = N // BN

    def body(h_ref, w_ref, y_ref):
        y_ref[...] = jnp.dot(h_ref[...], w_ref[0],
                             preferred_element_type=jnp.float32)

    return pl.pallas_call(
        body,
        grid=(NB, NC, R),
        in_specs=[
            pl.BlockSpec((BN, d_in), lambda nb, c, r: (nb, 0)),
            pl.BlockSpec((1, d_in, HC), lambda nb, c, r: (r, 0, c)),
        ],
        out_specs=pl.BlockSpec((BN, HC),
                               lambda nb, c, r: (c * R * NB + r * NB + nb, 0)),
        out_shape=jax.ShapeDtypeStruct((YROWS, HC), jnp.float32),
    )(h, W)


def _sc_agg(y, gidx, dst, w):
    EPT = EPAD // NS
    RPT = NPAD // NS
    ZR = 64

    @functools.partial(
        pl.kernel, mesh=_mesh(),
        out_type=jax.ShapeDtypeStruct((NC, NPAD, HC), jnp.float32),
        scratch_types=[
            pltpu.VMEM_SHARED((NPAD, HC), jnp.float32),
            pltpu.VMEM((WIN2, HC), jnp.float32),
            pltpu.VMEM((WIN2, HC), jnp.float32),
            pltpu.VMEM((CH2,), jnp.int32),
            pltpu.VMEM((CH2,), jnp.int32),
            pltpu.VMEM((CH2,), jnp.float32),
            pltpu.VMEM((WIN2,), jnp.int32),
            pltpu.VMEM((WIN2,), jnp.int32),
            pltpu.VMEM((WIN2,), jnp.int32),
            pltpu.VMEM((WIN2,), jnp.int32),
            pltpu.VMEM((ZR, HC), jnp.float32),
            pltpu.SemaphoreType.DMA,
            pltpu.SemaphoreType.DMA,
        ])
    def body(y_hbm, gidx_hbm, dst_hbm, w_hbm, msg_hbm,
             acc, rows_a, rows_b, gidx_big, dst_big, w_big,
             gidx_a, gidx_b, dst_a, dst_b, zbuf, sem_a, sem_b):
        c = lax.axis_index("c")
        s = lax.axis_index("s")

        @pl.loop(0, ZR)
        def _(i):
            for q in range(HC // 16):
                zbuf[i, pl.ds(q * 16, 16)] = jnp.zeros((16,), jnp.float32)

        @pl.loop(0, RPT // ZR)
        def _(k):
            pltpu.sync_copy(zbuf, acc.at[pl.ds(s * RPT + k * ZR, ZR)])

        plsc.subcore_barrier()
        off = c * (R * N)
        NWC = CH2 // WIN2

        def start(i, gidx_w, dst_w, rows, sem):
            @pl.loop(0, WIN2 // 16)
            def _(q):
                gidx_w[pl.ds(q * 16, 16)] = (
                    gidx_big[pl.ds(i * WIN2 + q * 16, 16)] + off)
                dst_w[pl.ds(q * 16, 16)] = (
                    dst_big[pl.ds(i * WIN2 + q * 16, 16)])
            pltpu.async_copy(y_hbm.at[gidx_w], rows, sem)

        def finish(i, gidx_w, dst_w, rows, sem):
            pltpu.make_async_copy(y_hbm.at[gidx_w], rows, sem).wait()

            @pl.loop(0, WIN2 // 16)
            def _(jj):
                wv = w_big[pl.ds(i * WIN2 + jj * 16, 16)]
                for l in range(16):
                    wj = wv[l]
                    for q in range(HC // 16):
                        rows[jj * 16 + l, pl.ds(q * 16, 16)] = (
                            rows[jj * 16 + l, pl.ds(q * 16, 16)] * wj)

            pltpu.sync_copy(rows, acc.at[dst_w], add=True)

        @pl.loop(0, EPT // CH2)
        def _(o):
            base = s * EPT + o * CH2
            pltpu.sync_copy(gidx_hbm.at[pl.ds(base, CH2)], gidx_big)
            pltpu.sync_copy(dst_hbm.at[pl.ds(base, CH2)], dst_big)
            pltpu.sync_copy(w_hbm.at[pl.ds(base, CH2)], w_big)

            start(0, gidx_a, dst_a, rows_a, sem_a)

            @pl.loop(0, NWC // 2)
            def _(k):
                i0 = 2 * k
                start(i0 + 1, gidx_b, dst_b, rows_b, sem_b)
                finish(i0, gidx_a, dst_a, rows_a, sem_a)

                @pl.when(i0 + 2 < NWC)
                def _():
                    start(i0 + 2, gidx_a, dst_a, rows_a, sem_a)

                finish(i0 + 1, gidx_b, dst_b, rows_b, sem_b)

        plsc.subcore_barrier()

        @pl.loop(0, RPT // ZR)
        def _(k):
            r0 = s * RPT + k * ZR
            pltpu.sync_copy(acc.at[pl.ds(r0, ZR)],
                            msg_hbm.at[c, pl.ds(r0, ZR)])

    return body(y, gidx, dst, w)


def _tc_out(h, root, b, msg):
    d_in = h.shape[1]
    BN = 1000
    NB = N // BN

    def body(h_ref, root_ref, b_ref, msg_ref, z_ref):
        t = jnp.dot(h_ref[...], root_ref[...],
                    preferred_element_type=jnp.float32)
        z_ref[...] = jnp.maximum(t + b_ref[0] + msg_ref[0], 0.0)

    return pl.pallas_call(
        body,
        grid=(NB, NC),
        in_specs=[
            pl.BlockSpec((BN, d_in), lambda nb, c: (nb, 0)),
            pl.BlockSpec((d_in, HC), lambda nb, c: (0, c)),
            pl.BlockSpec((1, 1, HC), lambda nb, c: (c, 0, 0)),
            pl.BlockSpec((1, BN, HC), lambda nb, c: (c, nb, 0)),
        ],
        out_specs=pl.BlockSpec((BN, HC), lambda nb, c: (nb, c)),
        out_shape=jax.ShapeDtypeStruct((N, DH), jnp.float32),
    )(h, root, b.reshape(NC, 1, HC), msg)


def kernel(x, edge_index, edge_type, W1, root1, b1, W2, root2, b2):
    src = edge_index[0].astype(jnp.int32)
    dst = edge_index[1].astype(jnp.int32)
    typ = edge_type.astype(jnp.int32)

    gidx, sidx = _prep0(src, dst, typ)
    cntp = _sc_count(sidx)
    inv = _prep1(cntp)
    w = _sc_w(inv, sidx)

    padlen = EPAD - E
    gidx_p = jnp.concatenate(
        [gidx, jnp.arange(padlen, dtype=jnp.int32) % (R * N)])
    dst_p = jnp.concatenate([dst, jnp.full((padlen,), N, jnp.int32)])
    w_p = jnp.concatenate([w, jnp.zeros((padlen,), jnp.float32)])

    y1 = _y_transform(x, W1)
    msg1 = _sc_agg(y1, gidx_p, dst_p, w_p)
    z = _tc_out(x, root1, b1, msg1)

    y2 = _y_transform(z, W2)
    msg2 = _sc_agg(y2, gidx_p, dst_p, w_p)
    out = _tc_out(z, root2, b2, msg2)
    return out


# speedup vs baseline: 1.1181x; 1.0390x over previous
import functools

import jax
import jax.numpy as jnp
from jax import lax
from jax.experimental import pallas as pl
from jax.experimental.pallas import tpu as pltpu
from jax.experimental.pallas import tpu_sc as plsc

N = 10000
E = 320000
R = 8
DH = 256
HC = 128
NC = 2
NS = 16
NW = NC * NS
TPW = 5120
CNT_PAD = NS * TPW
YROWS = NC * R * N

CH = 2000
WIN = 80
NPAD = 10240
EPAD = 322560
CH2 = 2016
WIN2 = 96


def _mesh():
    return plsc.VectorSubcoreMesh(
        core_axis_name="c", subcore_axis_name="s",
        num_cores=NC, num_subcores=NS)


def _prep0_body(src_ref, dst_ref, typ_ref, gidx_ref, sidx_ref):
    gidx_ref[...] = typ_ref[...] * N + src_ref[...]
    sidx_ref[...] = dst_ref[...] * R + typ_ref[...]


def _prep0(src, dst, typ):
    shp = (E // 128, 128)
    gidx, sidx = pl.pallas_call(
        _prep0_body,
        out_shape=(jax.ShapeDtypeStruct(shp, jnp.int32),
                   jax.ShapeDtypeStruct(shp, jnp.int32)),
    )(src.reshape(shp), dst.reshape(shp), typ.reshape(shp))
    return gidx.reshape(-1), sidx.reshape(-1)


def _sc_count(sidx):
    EPW = E // NW

    @functools.partial(
        pl.kernel, mesh=_mesh(),
        out_type=jax.ShapeDtypeStruct((NC, CNT_PAD), jnp.float32),
        scratch_types=[
            pltpu.VMEM_SHARED((CNT_PAD,), jnp.float32),
            pltpu.VMEM((TPW,), jnp.float32),
            pltpu.VMEM((CH,), jnp.int32),
            pltpu.VMEM((WIN,), jnp.int32),
            pltpu.VMEM((WIN,), jnp.float32),
        ])
    def body(sidx_hbm, out_hbm, cnt_sh, zbuf, sidx_big, sidx_w, ones_v):
        c = lax.axis_index("c")
        s = lax.axis_index("s")
        wid = s * NC + c

        @pl.loop(0, TPW // 16)
        def _(i):
            zbuf[pl.ds(i * 16, 16)] = jnp.zeros((16,), jnp.float32)

        @pl.loop(0, WIN // 16)
        def _(i):
            ones_v[pl.ds(i * 16, 16)] = jnp.ones((16,), jnp.float32)

        pltpu.sync_copy(zbuf, cnt_sh.at[pl.ds(s * TPW, TPW)])
        plsc.subcore_barrier()

        @pl.loop(0, EPW // CH)
        def _(o):
            base = wid * EPW + o * CH
            pltpu.sync_copy(sidx_hbm.at[pl.ds(base, CH)], sidx_big)

            @pl.loop(0, CH // WIN)
            def _(i):
                @pl.loop(0, WIN // 16)
                def _(q):
                    sidx_w[pl.ds(q * 16, 16)] = (
                        sidx_big[pl.ds(i * WIN + q * 16, 16)])
                pltpu.sync_copy(ones_v, cnt_sh.at[sidx_w], add=True)

        plsc.subcore_barrier()
        pltpu.sync_copy(cnt_sh.at[pl.ds(s * TPW, TPW)],
                        out_hbm.at[c, pl.ds(s * TPW, TPW)])

    return body(sidx)


def _prep1_body(cnt_ref, inv_ref):
    tot = cnt_ref[0] + cnt_ref[1]
    inv_ref[...] = 1.0 / jnp.maximum(tot, 1.0)


def _prep1(cntp):
    out = pl.pallas_call(
        _prep1_body,
        out_shape=jax.ShapeDtypeStruct((CNT_PAD // 128, 128), jnp.float32),
    )(cntp.reshape(NC, CNT_PAD // 128, 128))
    return out.reshape(-1)


def _sc_w(inv, sidx):
    EPW = E // NW

    @functools.partial(
        pl.kernel, mesh=_mesh(),
        out_type=jax.ShapeDtypeStruct((E,), jnp.float32),
        scratch_types=[
            pltpu.VMEM((CH,), jnp.int32),
            pltpu.VMEM((CH,), jnp.float32),
            pltpu.VMEM((WIN,), jnp.int32),
            pltpu.VMEM((WIN,), jnp.int32),
            pltpu.VMEM((WIN,), jnp.float32),
            pltpu.VMEM((WIN,), jnp.float32),
            pltpu.SemaphoreType.DMA,
            pltpu.SemaphoreType.DMA,
        ])
    def body(inv_hbm, sidx_hbm, w_hbm, sidx_big, w_big,
             sidx_wa, sidx_wb, w_wa, w_wb, sem_a, sem_b):
        c = lax.axis_index("c")
        s = lax.axis_index("s")
        wid = s * NC + c
        NWC = CH // WIN

        def start(i, sidx_w, w_win, sem):
            @pl.loop(0, WIN // 16)
            def _(q):
                sidx_w[pl.ds(q * 16, 16)] = (
                    sidx_big[pl.ds(i * WIN + q * 16, 16)])
            pltpu.async_copy(inv_hbm.at[sidx_w], w_win, sem)

        def finish(i, sidx_w, w_win, sem):
            pltpu.make_async_copy(inv_hbm.at[sidx_w], w_win, sem).wait()

            @pl.loop(0, WIN // 16)
            def _(q):
                w_big[pl.ds(i * WIN + q * 16, 16)] = (
                    w_win[pl.ds(q * 16, 16)])

        @pl.loop(0, EPW // CH)
        def _(o):
            base = wid * EPW + o * CH
            pltpu.sync_copy(sidx_hbm.at[pl.ds(base, CH)], sidx_big)

            start(0, sidx_wa, w_wa, sem_a)

            @pl.loop(0, (NWC - 1) // 2)
            def _(k):
                i0 = 2 * k
                start(i0 + 1, sidx_wb, w_wb, sem_b)
                finish(i0, sidx_wa, w_wa, sem_a)
                start(i0 + 2, sidx_wa, w_wa, sem_a)
                finish(i0 + 1, sidx_wb, w_wb, sem_b)

            finish(NWC - 1, sidx_wa, w_wa, sem_a)
            pltpu.sync_copy(w_big, w_hbm.at[pl.ds(base, CH)])

    return body(inv, sidx)


def _y_transform(h, W):
    d_in = h.shape[1]
    BN = 1000
    NB = N // BN

    def body(h_ref, w_ref, y_ref):
        y_ref[...] = jnp.dot(h_ref[...], w_ref[0],
                             preferred_element_type=jnp.float32)

    return pl.pallas_call(
        body,
        grid=(NB, NC, R),
        in_specs=[
            pl.BlockSpec((BN, d_in), lambda nb, c, r: (nb, 0)),
            pl.BlockSpec((1, d_in, HC), lambda nb, c, r: (r, 0, c)),
        ],
        out_specs=pl.BlockSpec((BN, HC),
                               lambda nb, c, r: (c * R * NB + r * NB + nb, 0)),
        out_shape=jax.ShapeDtypeStruct((YROWS, HC), jnp.float32),
    )(h, W)


def _sc_agg(y, gidx, dst, w):
    EPT = EPAD // NS
    RPT = NPAD // NS
    ZR = 32

    @functools.partial(
        pl.kernel, mesh=_mesh(),
        out_type=jax.ShapeDtypeStruct((NC, NPAD, HC), jnp.float32),
        scratch_types=[
            pltpu.VMEM_SHARED((NPAD, HC), jnp.float32),
            pltpu.VMEM((WIN2, HC), jnp.float32),
            pltpu.VMEM((WIN2, HC), jnp.float32),
            pltpu.VMEM((WIN2, HC), jnp.float32),
            pltpu.VMEM((CH2,), jnp.int32),
            pltpu.VMEM((CH2,), jnp.int32),
            pltpu.VMEM((CH2,), jnp.float32),
            pltpu.VMEM((WIN2,), jnp.int32),
            pltpu.VMEM((WIN2,), jnp.int32),
            pltpu.VMEM((WIN2,), jnp.int32),
            pltpu.VMEM((WIN2,), jnp.int32),
            pltpu.VMEM((WIN2,), jnp.int32),
            pltpu.VMEM((WIN2,), jnp.int32),
            pltpu.VMEM((ZR, HC), jnp.float32),
            pltpu.SemaphoreType.DMA,
            pltpu.SemaphoreType.DMA,
            pltpu.SemaphoreType.DMA,
            pltpu.SemaphoreType.DMA,
            pltpu.SemaphoreType.DMA,
            pltpu.SemaphoreType.DMA,
        ])
    def body(y_hbm, gidx_hbm, dst_hbm, w_hbm, msg_hbm,
             acc, rows_a, rows_b, rows_c, gidx_big, dst_big, w_big,
             gidx_a, gidx_b, gidx_c, dst_a, dst_b, dst_c, zbuf,
             gsem_a, gsem_b, gsem_c, ssem_a, ssem_b, ssem_c):
        c = lax.axis_index("c")
        s = lax.axis_index("s")

        @pl.loop(0, ZR)
        def _(i):
            for q in range(HC // 16):
                zbuf[i, pl.ds(q * 16, 16)] = jnp.zeros((16,), jnp.float32)

        @pl.loop(0, RPT // ZR)
        def _(k):
            pltpu.sync_copy(zbuf, acc.at[pl.ds(s * RPT + k * ZR, ZR)])

        plsc.subcore_barrier()
        off = c * (R * N)
        NWC = CH2 // WIN2

        A = (gidx_a, dst_a, rows_a, gsem_a, ssem_a)
        B = (gidx_b, dst_b, rows_b, gsem_b, ssem_b)
        C = (gidx_c, dst_c, rows_c, gsem_c, ssem_c)

        def start_g(i, buf):
            gidx_w, dst_w, rows, gsem, _ = buf

            @pl.loop(0, WIN2 // 16)
            def _(q):
                gidx_w[pl.ds(q * 16, 16)] = (
                    gidx_big[pl.ds(i * WIN2 + q * 16, 16)] + off)
                dst_w[pl.ds(q * 16, 16)] = (
                    dst_big[pl.ds(i * WIN2 + q * 16, 16)])
            pltpu.async_copy(y_hbm.at[gidx_w], rows, gsem)

        def scale_sc(i, buf):
            gidx_w, dst_w, rows, gsem, ssem = buf
            pltpu.make_async_copy(y_hbm.at[gidx_w], rows, gsem).wait()

            @pl.loop(0, WIN2 // 16)
            def _(jj):
                wv = w_big[pl.ds(i * WIN2 + jj * 16, 16)]
                for l in range(16):
                    wj = wv[l]
                    for q in range(HC // 16):
                        rows[jj * 16 + l, pl.ds(q * 16, 16)] = (
                            rows[jj * 16 + l, pl.ds(q * 16, 16)] * wj)

            pltpu.async_copy(rows, acc.at[dst_w], ssem, add=True)

        def wait_sc(buf):
            _, dst_w, rows, _, ssem = buf
            pltpu.make_async_copy(rows, acc.at[dst_w], ssem).wait()

        @pl.loop(0, EPT // CH2)
        def _(o):
            base = s * EPT + o * CH2
            pltpu.sync_copy(gidx_hbm.at[pl.ds(base, CH2)], gidx_big)
            pltpu.sync_copy(dst_hbm.at[pl.ds(base, CH2)], dst_big)
            pltpu.sync_copy(w_hbm.at[pl.ds(base, CH2)], w_big)

            start_g(0, A)
            start_g(1, B)

            @pl.loop(0, NWC // 3)
            def _(k):
                i0 = 3 * k

                @pl.when(k > 0)
                def _():
                    wait_sc(C)

                start_g(i0 + 2, C)
                scale_sc(i0, A)
                scale_sc(i0 + 1, B)
                wait_sc(A)

                @pl.when(i0 + 3 < NWC)
                def _():
                    start_g(i0 + 3, A)

                scale_sc(i0 + 2, C)
                wait_sc(B)

                @pl.when(i0 + 4 < NWC)
                def _():
                    start_g(i0 + 4, B)

            wait_sc(C)

        plsc.subcore_barrier()

        @pl.loop(0, RPT // ZR)
        def _(k):
            r0 = s * RPT + k * ZR
            pltpu.sync_copy(acc.at[pl.ds(r0, ZR)],
                            msg_hbm.at[c, pl.ds(r0, ZR)])

    return body(y, gidx, dst, w)


def _tc_out(h, root, b, msg):
    d_in = h.shape[1]
    BN = 1000
    NB = N // BN

    def body(h_ref, root_ref, b_ref, msg_ref, z_ref):
        t = jnp.dot(h_ref[...], root_ref[...],
                    preferred_element_type=jnp.float32)
        z_ref[...] = jnp.maximum(t + b_ref[0] + msg_ref[0], 0.0)

    return pl.pallas_call(
        body,
        grid=(NB, NC),
        in_specs=[
            pl.BlockSpec((BN, d_in), lambda nb, c: (nb, 0)),
            pl.BlockSpec((d_in, HC), lambda nb, c: (0, c)),
            pl.BlockSpec((1, 1, HC), lambda nb, c: (c, 0, 0)),
            pl.BlockSpec((1, BN, HC), lambda nb, c: (c, nb, 0)),
        ],
        out_specs=pl.BlockSpec((BN, HC), lambda nb, c: (nb, c)),
        out_shape=jax.ShapeDtypeStruct((N, DH), jnp.float32),
    )(h, root, b.reshape(NC, 1, HC), msg)


def kernel(x, edge_index, edge_type, W1, root1, b1, W2, root2, b2):
    src = edge_index[0].astype(jnp.int32)
    dst = edge_index[1].astype(jnp.int32)
    typ = edge_type.astype(jnp.int32)

    gidx, sidx = _prep0(src, dst, typ)
    cntp = _sc_count(sidx)
    inv = _prep1(cntp)
    w = _sc_w(inv, sidx)

    padlen = EPAD - E
    gidx_p = jnp.concatenate(
        [gidx, jnp.arange(padlen, dtype=jnp.int32) % (R * N)])
    dst_p = jnp.concatenate([dst, jnp.full((padlen,), N, jnp.int32)])
    w_p = jnp.concatenate([w, jnp.zeros((padlen,), jnp.float32)])

    y1 = _y_transform(x, W1)
    msg1 = _sc_agg(y1, gidx_p, dst_p, w_p)
    z = _tc_out(x, root1, b1, msg1)

    y2 = _y_transform(z, W2)
    msg2 = _sc_agg(y2, gidx_p, dst_p, w_p)
    out = _tc_out(z, root2, b2, msg2)
    return out


# speedup vs baseline: 1.1182x; 1.0001x over previous
import functools

import jax
import jax.numpy as jnp
from jax import lax
from jax.experimental import pallas as pl
from jax.experimental.pallas import tpu as pltpu
from jax.experimental.pallas import tpu_sc as plsc

N = 10000
E = 320000
R = 8
DH = 256
HC = 128
NC = 2
NS = 16
NW = NC * NS
TPW = 5120
CNT_PAD = NS * TPW
YROWS = NC * R * N

CH = 2000
WIN = 80
NPAD = 10240
EPAD = 322560
CH2 = 2016
WIN2 = 96


def _mesh():
    return plsc.VectorSubcoreMesh(
        core_axis_name="c", subcore_axis_name="s",
        num_cores=NC, num_subcores=NS)


def _prep0_body(src_ref, dst_ref, typ_ref, gidx_ref, sidx_ref):
    gidx_ref[...] = typ_ref[...] * N + src_ref[...]
    sidx_ref[...] = dst_ref[...] * R + typ_ref[...]


def _prep0(src, dst, typ):
    shp = (E // 128, 128)
    gidx, sidx = pl.pallas_call(
        _prep0_body,
        out_shape=(jax.ShapeDtypeStruct(shp, jnp.int32),
                   jax.ShapeDtypeStruct(shp, jnp.int32)),
    )(src.reshape(shp), dst.reshape(shp), typ.reshape(shp))
    return gidx.reshape(-1), sidx.reshape(-1)


def _sc_count(sidx):
    EPW = E // NW

    @functools.partial(
        pl.kernel, mesh=_mesh(),
        out_type=jax.ShapeDtypeStruct((NC, CNT_PAD), jnp.float32),
        scratch_types=[
            pltpu.VMEM_SHARED((CNT_PAD,), jnp.float32),
            pltpu.VMEM((TPW,), jnp.float32),
            pltpu.VMEM((CH,), jnp.int32),
            pltpu.VMEM((WIN,), jnp.int32),
            pltpu.VMEM((WIN,), jnp.float32),
        ])
    def body(sidx_hbm, out_hbm, cnt_sh, zbuf, sidx_big, sidx_w, ones_v):
        c = lax.axis_index("c")
        s = lax.axis_index("s")
        wid = s * NC + c

        @pl.loop(0, TPW // 16)
        def _(i):
            zbuf[pl.ds(i * 16, 16)] = jnp.zeros((16,), jnp.float32)

        @pl.loop(0, WIN // 16)
        def _(i):
            ones_v[pl.ds(i * 16, 16)] = jnp.ones((16,), jnp.float32)

        pltpu.sync_copy(zbuf, cnt_sh.at[pl.ds(s * TPW, TPW)])
        plsc.subcore_barrier()

        @pl.loop(0, EPW // CH)
        def _(o):
            base = wid * EPW + o * CH
            pltpu.sync_copy(sidx_hbm.at[pl.ds(base, CH)], sidx_big)

            @pl.loop(0, CH // WIN)
            def _(i):
                @pl.loop(0, WIN // 16)
                def _(q):
                    sidx_w[pl.ds(q * 16, 16)] = (
                        sidx_big[pl.ds(i * WIN + q * 16, 16)])
                pltpu.sync_copy(ones_v, cnt_sh.at[sidx_w], add=True)

        plsc.subcore_barrier()
        pltpu.sync_copy(cnt_sh.at[pl.ds(s * TPW, TPW)],
                        out_hbm.at[c, pl.ds(s * TPW, TPW)])

    return body(sidx)


def _prep1_body(cnt_ref, inv_ref):
    tot = cnt_ref[0] + cnt_ref[1]
    inv_ref[...] = 1.0 / jnp.maximum(tot, 1.0)


def _prep1(cntp):
    out = pl.pallas_call(
        _prep1_body,
        out_shape=jax.ShapeDtypeStruct((CNT_PAD // 128, 128), jnp.float32),
    )(cntp.reshape(NC, CNT_PAD // 128, 128))
    return out.reshape(-1)


def _sc_w(inv, sidx):
    EPW = E // NW

    @functools.partial(
        pl.kernel, mesh=_mesh(),
        out_type=jax.ShapeDtypeStruct((E,), jnp.float32),
        scratch_types=[
            pltpu.VMEM((CH,), jnp.int32),
            pltpu.VMEM((CH // WIN, WIN), jnp.int32),
            pltpu.VMEM((CH,), jnp.float32),
            pltpu.SemaphoreType.DMA,
        ])
    def body(inv_hbm, sidx_hbm, w_hbm, sidx_big, sidx_wf, w_wf, sem):
        c = lax.axis_index("c")
        s = lax.axis_index("s")
        wid = s * NC + c
        NWC = CH // WIN

        @pl.loop(0, EPW // CH)
        def _(o):
            base = wid * EPW + o * CH
            pltpu.sync_copy(sidx_hbm.at[pl.ds(base, CH)], sidx_big)

            @pl.loop(0, NWC)
            def _(j):
                @pl.loop(0, WIN // 16)
                def _(q):
                    sidx_wf[j, pl.ds(q * 16, 16)] = (
                        sidx_big[pl.ds(j * WIN + q * 16, 16)])
                pltpu.async_copy(inv_hbm.at[sidx_wf.at[j]],
                                 w_wf.at[pl.ds(j * WIN, WIN)], sem)

            @pl.loop(0, NWC)
            def _(j):
                pltpu.make_async_copy(inv_hbm.at[sidx_wf.at[j]],
                                      w_wf.at[pl.ds(j * WIN, WIN)],
                                      sem).wait()

            pltpu.sync_copy(w_wf, w_hbm.at[pl.ds(base, CH)])

    return body(inv, sidx)


def _y_transform(h, W):
    d_in = h.shape[1]
    BN = 1000
    NB = N // BN

    def body(h_ref, w_ref, y_ref):
        y_ref[...] = jnp.dot(h_ref[...], w_ref[0],
                             preferred_element_type=jnp.float32)

    return pl.pallas_call(
        body,
        grid=(NB, NC, R),
        in_specs=[
            pl.BlockSpec((BN, d_in), lambda nb, c, r: (nb, 0)),
            pl.BlockSpec((1, d_in, HC), lambda nb, c, r: (r, 0, c)),
        ],
        out_specs=pl.BlockSpec((BN, HC),
                               lambda nb, c, r: (c * R * NB + r * NB + nb, 0)),
        out_shape=jax.ShapeDtypeStruct((YROWS, HC), jnp.float32),
    )(h, W)


def _sc_agg(y, gidx, dst, w):
    EPT = EPAD // NS
    RPT = NPAD // NS
    ZR = 32

    @functools.partial(
        pl.kernel, mesh=_mesh(),
        out_type=jax.ShapeDtypeStruct((NC, NPAD, HC), jnp.float32),
        scratch_types=[
            pltpu.VMEM_SHARED((NPAD, HC), jnp.float32),
            pltpu.VMEM((WIN2, HC), jnp.float32),
            pltpu.VMEM((WIN2, HC), jnp.float32),
            pltpu.VMEM((WIN2, HC), jnp.float32),
            pltpu.VMEM((CH2,), jnp.int32),
            pltpu.VMEM((CH2,), jnp.int32),
            pltpu.VMEM((CH2,), jnp.float32),
            pltpu.VMEM((WIN2,), jnp.int32),
            pltpu.VMEM((WIN2,), jnp.int32),
            pltpu.VMEM((WIN2,), jnp.int32),
            pltpu.VMEM((WIN2,), jnp.int32),
            pltpu.VMEM((WIN2,), jnp.int32),
            pltpu.VMEM((WIN2,), jnp.int32),
            pltpu.VMEM((ZR, HC), jnp.float32),
            pltpu.SemaphoreType.DMA,
            pltpu.SemaphoreType.DMA,
            pltpu.SemaphoreType.DMA,
            pltpu.SemaphoreType.DMA,
            pltpu.SemaphoreType.DMA,
            pltpu.SemaphoreType.DMA,
        ])
    def body(y_hbm, gidx_hbm, dst_hbm, w_hbm, msg_hbm,
             acc, rows_a, rows_b, rows_c, gidx_big, dst_big, w_big,
             gidx_a, gidx_b, gidx_c, dst_a, dst_b, dst_c, zbuf,
             gsem_a, gsem_b, gsem_c, ssem_a, ssem_b, ssem_c):
        c = lax.axis_index("c")
        s = lax.axis_index("s")

        @pl.loop(0, ZR)
        def _(i):
            for q in range(HC // 16):
                zbuf[i, pl.ds(q * 16, 16)] = jnp.zeros((16,), jnp.float32)

        @pl.loop(0, RPT // ZR)
        def _(k):
            pltpu.sync_copy(zbuf, acc.at[pl.ds(s * RPT + k * ZR, ZR)])

        plsc.subcore_barrier()
        off = c * (R * N)
        NWC = CH2 // WIN2

        A = (gidx_a, dst_a, rows_a, gsem_a, ssem_a)
        B = (gidx_b, dst_b, rows_b, gsem_b, ssem_b)
        C = (gidx_c, dst_c, rows_c, gsem_c, ssem_c)

        def start_g(i, buf):
            gidx_w, dst_w, rows, gsem, _ = buf

            @pl.loop(0, WIN2 // 16)
            def _(q):
                gidx_w[pl.ds(q * 16, 16)] = (
                    gidx_big[pl.ds(i * WIN2 + q * 16, 16)] + off)
                dst_w[pl.ds(q * 16, 16)] = (
                    dst_big[pl.ds(i * WIN2 + q * 16, 16)])
            pltpu.async_copy(y_hbm.at[gidx_w], rows, gsem)

        def scale_sc(i, buf):
            gidx_w, dst_w, rows, gsem, ssem = buf
            pltpu.make_async_copy(y_hbm.at[gidx_w], rows, gsem).wait()

            @pl.loop(0, WIN2 // 16)
            def _(jj):
                wv = w_big[pl.ds(i * WIN2 + jj * 16, 16)]
                for l in range(16):
                    wj = wv[l]
                    for q in range(HC // 16):
                        rows[jj * 16 + l, pl.ds(q * 16, 16)] = (
                            rows[jj * 16 + l, pl.ds(q * 16, 16)] * wj)

            pltpu.async_copy(rows, acc.at[dst_w], ssem, add=True)

        def wait_sc(buf):
            _, dst_w, rows, _, ssem = buf
            pltpu.make_async_copy(rows, acc.at[dst_w], ssem).wait()

        @pl.loop(0, EPT // CH2)
        def _(o):
            base = s * EPT + o * CH2
            pltpu.sync_copy(gidx_hbm.at[pl.ds(base, CH2)], gidx_big)
            pltpu.sync_copy(dst_hbm.at[pl.ds(base, CH2)], dst_big)
            pltpu.sync_copy(w_hbm.at[pl.ds(base, CH2)], w_big)

            start_g(0, A)
            start_g(1, B)

            @pl.loop(0, NWC // 3)
            def _(k):
                i0 = 3 * k

                @pl.when(k > 0)
                def _():
                    wait_sc(C)

                start_g(i0 + 2, C)
                scale_sc(i0, A)
                scale_sc(i0 + 1, B)
                wait_sc(A)

                @pl.when(i0 + 3 < NWC)
                def _():
                    start_g(i0 + 3, A)

                scale_sc(i0 + 2, C)
                wait_sc(B)

                @pl.when(i0 + 4 < NWC)
                def _():
                    start_g(i0 + 4, B)

            wait_sc(C)

        plsc.subcore_barrier()

        @pl.loop(0, RPT // ZR)
        def _(k):
            r0 = s * RPT + k * ZR
            pltpu.sync_copy(acc.at[pl.ds(r0, ZR)],
                            msg_hbm.at[c, pl.ds(r0, ZR)])

    return body(y, gidx, dst, w)


def _tc_out(h, root, b, msg):
    d_in = h.shape[1]
    BN = 1000
    NB = N // BN

    def body(h_ref, root_ref, b_ref, msg_ref, z_ref):
        t = jnp.dot(h_ref[...], root_ref[...],
                    preferred_element_type=jnp.float32)
        z_ref[...] = jnp.maximum(t + b_ref[0] + msg_ref[0], 0.0)

    return pl.pallas_call(
        body,
        grid=(NB, NC),
        in_specs=[
            pl.BlockSpec((BN, d_in), lambda nb, c: (nb, 0)),
            pl.BlockSpec((d_in, HC), lambda nb, c: (0, c)),
            pl.BlockSpec((1, 1, HC), lambda nb, c: (c, 0, 0)),
            pl.BlockSpec((1, BN, HC), lambda nb, c: (c, nb, 0)),
        ],
        out_specs=pl.BlockSpec((BN, HC), lambda nb, c: (nb, c)),
        out_shape=jax.ShapeDtypeStruct((N, DH), jnp.float32),
    )(h, root, b.reshape(NC, 1, HC), msg)


def kernel(x, edge_index, edge_type, W1, root1, b1, W2, root2, b2):
    src = edge_index[0].astype(jnp.int32)
    dst = edge_index[1].astype(jnp.int32)
    typ = edge_type.astype(jnp.int32)

    gidx, sidx = _prep0(src, dst, typ)
    cntp = _sc_count(sidx)
    inv = _prep1(cntp)
    w = _sc_w(inv, sidx)

    padlen = EPAD - E
    gidx_p = jnp.concatenate(
        [gidx, jnp.arange(padlen, dtype=jnp.int32) % (R * N)])
    dst_p = jnp.concatenate([dst, jnp.full((padlen,), N, jnp.int32)])
    w_p = jnp.concatenate([w, jnp.zeros((padlen,), jnp.float32)])

    y1 = _y_transform(x, W1)
    msg1 = _sc_agg(y1, gidx_p, dst_p, w_p)
    z = _tc_out(x, root1, b1, msg1)

    y2 = _y_transform(z, W2)
    msg2 = _sc_agg(y2, gidx_p, dst_p, w_p)
    out = _tc_out(z, root2, b2, msg2)
    return out
